# baseline (speedup 1.0000x reference)
import functools

import jax
import jax.numpy as jnp
import numpy as np
from jax import lax
from jax.experimental import pallas as pl
from jax.experimental.pallas import tpu as pltpu
from jax.experimental.pallas import tpu_sc as plsc

NC = 2
NS = 16
NW = NC * NS
CHUNK = 64
INNER = 2
M0 = 80
M1 = 80


def _matmul_body(f_ref, w_ref, o_ref):
    o_ref[0] = jnp.dot(
        f_ref[...], w_ref[0], preferred_element_type=jnp.float32
    ).astype(jnp.bfloat16)


def _add_relu_body(a_ref, b_ref, o_ref):
    o_ref[...] = jnp.maximum(a_ref[0] + b_ref[0], 0.0)


@functools.partial(jax.jit, static_argnames=("npad", "c", "rpt"))
def _sc_gather_scatter(idx, twords, zeros, *, npad, c, rpt):
    mesh = plsc.VectorSubcoreMesh(
        core_axis_name="c", subcore_axis_name="s", num_cores=NC, num_subcores=NS
    )
    ngrp = c // 32

    def body(idx_hbm, twords_hbm, zeros_hbm, out_hbm,
             idx_v, braw_v, rows_v, accum_sh, isem, gsem, ssem):
        cid = lax.axis_index("c")
        sid = lax.axis_index("s")
        wid = cid * NS + sid
        pltpu.sync_copy(idx_hbm.at[wid, 0], idx_v.at[0])
        pltpu.sync_copy(zeros_hbm.at[pl.ds(sid * rpt, rpt)],
                        accum_sh.at[pl.ds(sid * rpt, rpt)])
        plsc.subcore_barrier()

        def widen(b):
            def row_body(r, carry):
                for g in range(ngrp):
                    x = braw_v[b, r, pl.ds(16 * g, 16)]
                    lo = plsc.bitcast(x << 16, jnp.float32)
                    hi = plsc.bitcast(x & jnp.int32(-65536), jnp.float32)
                    rows_v[b, r, pl.ds(32 * g, 16)] = lo
                    rows_v[b, r, pl.ds(32 * g + 16, 16)] = hi
                return carry
            lax.fori_loop(0, CHUNK, row_body, 0)

        def outer_pair(t2, carry):
            for p in range(2):
                t = 2 * t2 + p
                nxt = pltpu.async_copy(idx_hbm.at[wid, t + 1],
                                       idx_v.at[1 - p], isem)
                gd = [
                    pltpu.async_copy(twords_hbm.at[idx_v.at[p, 0, b]],
                                     braw_v.at[b], gsem)
                    for b in range(INNER)
                ]
                sd = []
                for b in range(INNER):
                    gd[b].wait()
                    widen(b)
                    sd.append(pltpu.async_copy(rows_v.at[b],
                                               accum_sh.at[idx_v.at[p, 1, b]],
                                               ssem, add=True))
                for d in sd:
                    d.wait()
                nxt.wait()
            return carry

        lax.fori_loop(0, lax.select(cid == 0, M0 // 2, M1 // 2),
                      outer_pair, 0)
        plsc.subcore_barrier()
        pltpu.sync_copy(accum_sh.at[pl.ds(sid * rpt, rpt)],
                        out_hbm.at[cid, pl.ds(sid * rpt, rpt)])

    return pl.kernel(
        body,
        out_type=jax.ShapeDtypeStruct((NC, npad, c), jnp.float32),
        mesh=mesh,
        compiler_params=pltpu.CompilerParams(
            needs_layout_passes=False, use_tc_tiling_on_sc=False),
        scratch_types=[
            pltpu.VMEM((2, 2, INNER, CHUNK), jnp.int32),
            pltpu.VMEM((INNER, CHUNK, c // 2), jnp.int32),
            pltpu.VMEM((INNER, CHUNK, c), jnp.float32),
            pltpu.VMEM_SHARED((npad, c), jnp.float32),
            pltpu.SemaphoreType.DMA,
            pltpu.SemaphoreType.DMA,
            pltpu.SemaphoreType.DMA,
        ],
    )(idx, twords, zeros)


def _interleave_perm(c):
    perm = np.empty((c,), np.int32)
    for g in range(c // 32):
        for j in range(16):
            perm[32 * g + 2 * j] = 32 * g + j
            perm[32 * g + 2 * j + 1] = 32 * g + 16 + j
    return perm


def kernel(feats, in_map, out_map, W):
    n, c_in = feats.shape
    k, e = in_map.shape
    c_out = W.shape[-1]

    in32 = in_map.astype(jnp.int32)
    out32 = out_map.astype(jnp.int32)
    perm = _interleave_perm(c_out)

    bn = 2000
    w_perm = W[:, :, jnp.asarray(perm)]
    tfeats = pl.pallas_call(
        _matmul_body,
        grid=(n // bn, k),
        in_specs=[
            pl.BlockSpec((bn, c_in), lambda ni, ki: (ni, 0)),
            pl.BlockSpec((1, c_in, c_out), lambda ni, ki: (ki, 0, 0)),
        ],
        out_specs=pl.BlockSpec((1, bn, c_out), lambda ni, ki: (ki, ni, 0)),
        out_shape=jax.ShapeDtypeStruct((k, n, c_out), jnp.bfloat16),
    )(feats, w_perm)
    twords = lax.bitcast_convert_type(
        tfeats.reshape(k * n, c_out // 2, 2), jnp.int32
    )

    etot = k * e
    blk = INNER * CHUNK
    ep0 = NS * M0 * blk
    ep1 = NS * M1 * blk
    ep = ep0 + ep1
    assert ep >= etot
    npad = ((n + NS + 127) // 128) * 128
    rpt = npad // NS

    offs = (jnp.arange(k, dtype=jnp.int32) * n)[:, None]
    fi = (in32 + offs).reshape(-1)
    fo = out32.reshape(-1)
    fi = jnp.concatenate([fi, jnp.zeros((ep - etot,), jnp.int32)])
    fo = jnp.concatenate([fo, jnp.full((ep - etot,), n, jnp.int32)])

    def _pack(x):
        p0 = x[:ep0].reshape(NS, M0, 1, INNER, CHUNK)
        p1 = x[ep0:].reshape(NS, M1, 1, INNER, CHUNK)
        pad0 = jnp.zeros((NS, 1, 1, INNER, CHUNK), jnp.int32)
        pad1 = jnp.zeros((NS, M0 - M1 + 1, 1, INNER, CHUNK), jnp.int32)
        return jnp.concatenate(
            [jnp.concatenate([p0, pad0], axis=1),
             jnp.concatenate([p1, pad1], axis=1)], axis=0)

    idx = jnp.concatenate([_pack(fi), _pack(fo)], axis=2)

    zeros = jnp.zeros((npad, c_out), jnp.float32)
    partials = _sc_gather_scatter(
        idx, twords, zeros, npad=npad, c=c_out, rpt=rpt
    )

    out = pl.pallas_call(
        _add_relu_body,
        grid=(1,),
        in_specs=[
            pl.BlockSpec((1, npad, c_out), lambda i: (0, 0, 0)),
            pl.BlockSpec((1, npad, c_out), lambda i: (1, 0, 0)),
        ],
        out_specs=pl.BlockSpec((npad, c_out), lambda i: (0, 0)),
        out_shape=jax.ShapeDtypeStruct((npad, c_out), jnp.float32),
    )(partials, partials)
    return out[:n]


# speedup vs baseline: 1.2482x; 1.2482x over previous
import functools

import jax
import jax.numpy as jnp
from jax import lax
from jax.experimental import pallas as pl
from jax.experimental.pallas import tpu as pltpu
from jax.experimental.pallas import tpu_sc as plsc

NC = 2
NS = 16
NW = NC * NS
CHUNK = 128
INNER = 2
NG = 3
M0 = 18
M1 = 10


def _matmul_body(f_ref, w_ref, o_ref):
    o_ref[0] = jnp.dot(f_ref[...], w_ref[0], preferred_element_type=jnp.float32)


def _sum_relu_body(*refs):
    o_ref = refs[-1]
    acc = refs[0][0]
    for r in refs[1:-1]:
        acc = acc + r[0]
    o_ref[...] = jnp.maximum(acc, 0.0)


@functools.partial(jax.jit, static_argnames=("npad", "c", "rpt"))
def _sc_gather_scatter(idx, tflat, zeros, *, npad, c, rpt):
    mesh = plsc.VectorSubcoreMesh(
        core_axis_name="c", subcore_axis_name="s", num_cores=NC, num_subcores=NS
    )

    def body(idx_hbm, tfeats_hbm, zeros_hbm, out_hbm,
             idx_v, rows_v, accum_sh, isem, gsem, ssem):
        cid = lax.axis_index("c")
        sid = lax.axis_index("s")
        wid = cid * NS + sid
        pltpu.sync_copy(idx_hbm.at[wid, 0], idx_v.at[0])
        pltpu.sync_copy(zeros_hbm.at[pl.ds(sid * rpt, rpt)],
                        accum_sh.at[pl.ds(sid * rpt, rpt)])
        plsc.subcore_barrier()

        def outer_pair(t2, carry):
            for p in range(2):
                t = 2 * t2 + p
                nxt = pltpu.async_copy(idx_hbm.at[wid, t + 1],
                                       idx_v.at[1 - p], isem)
                gd = [
                    pltpu.async_copy(tfeats_hbm.at[idx_v.at[p, 0, b]],
                                     rows_v.at[b], gsem)
                    for b in range(INNER)
                ]
                sd = []
                for b in range(INNER):
                    gd[b].wait()
                    sd.append(pltpu.async_copy(rows_v.at[b],
                                               accum_sh.at[idx_v.at[p, 1, b]],
                                               ssem, add=True))
                for d in sd:
                    d.wait()
                nxt.wait()
            return carry

        lax.fori_loop(0, lax.select(cid == 0, M0 // 2, M1 // 2),
                      outer_pair, 0)
        plsc.subcore_barrier()
        pltpu.sync_copy(accum_sh.at[pl.ds(sid * rpt, rpt)],
                        out_hbm.at[cid, pl.ds(sid * rpt, rpt)])

    return pl.kernel(
        body,
        out_type=jax.ShapeDtypeStruct((NC, npad, c), jnp.float32),
        mesh=mesh,
        scratch_types=[
            pltpu.VMEM((2, 2, INNER, CHUNK), jnp.int32),
            pltpu.VMEM((INNER, CHUNK, c), jnp.float32),
            pltpu.VMEM_SHARED((npad, c), jnp.float32),
            pltpu.SemaphoreType.DMA,
            pltpu.SemaphoreType.DMA,
            pltpu.SemaphoreType.DMA,
        ],
    )(idx, tflat, zeros)


def kernel(feats, in_map, out_map, W):
    n, c_in = feats.shape
    k, e = in_map.shape
    c_out = W.shape[-1]
    kg = k // NG
    assert kg * NG == k

    in32 = in_map.astype(jnp.int32)
    out32 = out_map.astype(jnp.int32)
    feats_bf = feats.astype(jnp.bfloat16)
    w_bf = W.astype(jnp.bfloat16)

    blk = INNER * CHUNK
    ep0 = NS * M0 * blk
    ep1 = NS * M1 * blk
    ep = ep0 + ep1
    eg = kg * e
    assert ep >= eg
    npad = ((n + NS + 127) // 128) * 128
    rpt = npad // NS
    zeros = jnp.zeros((npad, c_out), jnp.float32)
    bn = 2000

    def _pack(x):
        p0 = x[:ep0].reshape(NS, M0, 1, INNER, CHUNK)
        p1 = x[ep0:].reshape(NS, M1, 1, INNER, CHUNK)
        pad0 = jnp.zeros((NS, 1, 1, INNER, CHUNK), jnp.int32)
        pad1 = jnp.zeros((NS, M0 - M1 + 1, 1, INNER, CHUNK), jnp.int32)
        return jnp.concatenate(
            [jnp.concatenate([p0, pad0], axis=1),
             jnp.concatenate([p1, pad1], axis=1)], axis=0)

    offs = (jnp.arange(kg, dtype=jnp.int32) * n)[:, None]
    partials = []
    for g in range(NG):
        tfeats = pl.pallas_call(
            _matmul_body,
            grid=(n // bn, kg),
            in_specs=[
                pl.BlockSpec((bn, c_in), lambda ni, ki: (ni, 0)),
                pl.BlockSpec((1, c_in, c_out), lambda ni, ki: (ki, 0, 0)),
            ],
            out_specs=pl.BlockSpec((1, bn, c_out), lambda ni, ki: (ki, ni, 0)),
            out_shape=jax.ShapeDtypeStruct((kg, n, c_out), jnp.float32),
        )(feats_bf, w_bf[g * kg:(g + 1) * kg])
        tflat = tfeats.reshape(kg * n, c_out)

        fi = (in32[g * kg:(g + 1) * kg] + offs).reshape(-1)
        fo = out32[g * kg:(g + 1) * kg].reshape(-1)
        fi = jnp.concatenate([fi, jnp.zeros((ep - eg,), jnp.int32)])
        fo = jnp.concatenate([fo, jnp.full((ep - eg,), n, jnp.int32)])
        idx = jnp.concatenate([_pack(fi), _pack(fo)], axis=2)

        partials.append(_sc_gather_scatter(
            idx, tflat, zeros, npad=npad, c=c_out, rpt=rpt))

    ins = []
    in_specs = []
    for p in partials:
        for cc in range(NC):
            ins.append(p)
            in_specs.append(
                pl.BlockSpec((1, n, c_out), lambda i, cc=cc: (cc, 0, 0)))
    out = pl.pallas_call(
        _sum_relu_body,
        grid=(1,),
        in_specs=in_specs,
        out_specs=pl.BlockSpec((n, c_out), lambda i: (0, 0)),
        out_shape=jax.ShapeDtypeStruct((n, c_out), jnp.float32),
    )(*ins)
    return out


# speedup vs baseline: 2.9886x; 2.3943x over previous
import functools

import jax
import jax.numpy as jnp
from jax import lax
from jax.experimental import pallas as pl
from jax.experimental.pallas import tpu as pltpu
from jax.experimental.pallas import tpu_sc as plsc

NC = 2
NS = 16
NW = NC * NS
CHUNK = 64
INNER = 5
NG = 1
M0 = 42
M1 = 22


def _matmul_body(f_ref, w_ref, o_ref):
    o_ref[0] = jnp.dot(f_ref[...], w_ref[0], preferred_element_type=jnp.float32)


def _sum_relu_body(*refs):
    o_ref = refs[-1]
    acc = refs[0][0]
    for r in refs[1:-1]:
        acc = acc + r[0]
    o_ref[...] = jnp.maximum(acc, 0.0)


@functools.partial(jax.jit, static_argnames=("npad", "c", "rpt"))
def _sc_gather_scatter(idx, tflat, zeros, *, npad, c, rpt):
    mesh = plsc.VectorSubcoreMesh(
        core_axis_name="c", subcore_axis_name="s", num_cores=NC, num_subcores=NS
    )

    def body(idx_hbm, tfeats_hbm, zeros_hbm, out_hbm,
             idx_v, rows_v, accum_sh, isem, gsem, ssem):
        cid = lax.axis_index("c")
        sid = lax.axis_index("s")
        wid = cid * NS + sid
        pltpu.sync_copy(idx_hbm.at[wid, 0], idx_v.at[0])
        pltpu.sync_copy(zeros_hbm.at[pl.ds(sid * rpt, rpt)],
                        accum_sh.at[pl.ds(sid * rpt, rpt)])
        plsc.subcore_barrier()

        def outer_pair(t2, carry):
            for p in range(2):
                t = 2 * t2 + p
                nxt = pltpu.async_copy(idx_hbm.at[wid, t + 1],
                                       idx_v.at[1 - p], isem)
                gd = [
                    pltpu.async_copy(tfeats_hbm.at[idx_v.at[p, 0, b]],
                                     rows_v.at[b], gsem)
                    for b in range(INNER)
                ]
                sd = []
                for b in range(INNER):
                    gd[b].wait()
                    sd.append(pltpu.async_copy(rows_v.at[b],
                                               accum_sh.at[idx_v.at[p, 1, b]],
                                               ssem, add=True))
                for d in sd:
                    d.wait()
                nxt.wait()
            return carry

        lax.fori_loop(0, lax.select(cid == 0, M0 // 2, M1 // 2),
                      outer_pair, 0)
        plsc.subcore_barrier()
        pltpu.sync_copy(accum_sh.at[pl.ds(sid * rpt, rpt)],
                        out_hbm.at[cid, pl.ds(sid * rpt, rpt)])

    return pl.kernel(
        body,
        out_type=jax.ShapeDtypeStruct((NC, npad, c), jnp.float32),
        mesh=mesh,
        scratch_types=[
            pltpu.VMEM((2, 2, INNER, CHUNK), jnp.int32),
            pltpu.VMEM((INNER, CHUNK, c), jnp.float32),
            pltpu.VMEM_SHARED((npad, c), jnp.float32),
            pltpu.SemaphoreType.DMA,
            pltpu.SemaphoreType.DMA,
            pltpu.SemaphoreType.DMA,
        ],
    )(idx, tflat, zeros)


def kernel(feats, in_map, out_map, W):
    n, c_in = feats.shape
    k, e = in_map.shape
    c_out = W.shape[-1]
    kg = k // NG
    assert kg * NG == k

    in32 = in_map.astype(jnp.int32)
    out32 = out_map.astype(jnp.int32)
    feats_bf = feats.astype(jnp.bfloat16)
    w_bf = W.astype(jnp.bfloat16)

    blk = INNER * CHUNK
    ep0 = NS * M0 * blk
    ep1 = NS * M1 * blk
    ep = ep0 + ep1
    eg = kg * e
    assert ep >= eg
    npad = ((n + NS + 127) // 128) * 128
    rpt = npad // NS
    zeros = jnp.zeros((npad, c_out), jnp.float32)
    bn = 2000

    def _pack(x):
        p0 = x[:ep0].reshape(NS, M0, 1, INNER, CHUNK)
        p1 = x[ep0:].reshape(NS, M1, 1, INNER, CHUNK)
        pad0 = jnp.zeros((NS, 1, 1, INNER, CHUNK), jnp.int32)
        pad1 = jnp.zeros((NS, M0 - M1 + 1, 1, INNER, CHUNK), jnp.int32)
        return jnp.concatenate(
            [jnp.concatenate([p0, pad0], axis=1),
             jnp.concatenate([p1, pad1], axis=1)], axis=0)

    offs = (jnp.arange(kg, dtype=jnp.int32) * n)[:, None]
    partials = []
    for g in range(NG):
        tfeats = pl.pallas_call(
            _matmul_body,
            grid=(n // bn, kg),
            in_specs=[
                pl.BlockSpec((bn, c_in), lambda ni, ki: (ni, 0)),
                pl.BlockSpec((1, c_in, c_out), lambda ni, ki: (ki, 0, 0)),
            ],
            out_specs=pl.BlockSpec((1, bn, c_out), lambda ni, ki: (ki, ni, 0)),
            out_shape=jax.ShapeDtypeStruct((kg, n, c_out), jnp.float32),
        )(feats_bf, w_bf[g * kg:(g + 1) * kg])
        tflat = tfeats.reshape(kg * n, c_out)

        fi = (in32[g * kg:(g + 1) * kg] + offs).reshape(-1)
        fo = out32[g * kg:(g + 1) * kg].reshape(-1)
        fi = jnp.concatenate([fi, jnp.zeros((ep - eg,), jnp.int32)])
        fo = jnp.concatenate([fo, jnp.full((ep - eg,), n, jnp.int32)])
        idx = jnp.concatenate([_pack(fi), _pack(fo)], axis=2)

        partials.append(_sc_gather_scatter(
            idx, tflat, zeros, npad=npad, c=c_out, rpt=rpt))

    ins = []
    in_specs = []
    for p in partials:
        for cc in range(NC):
            ins.append(p)
            in_specs.append(
                pl.BlockSpec((1, n, c_out), lambda i, cc=cc: (cc, 0, 0)))
    out = pl.pallas_call(
        _sum_relu_body,
        grid=(1,),
        in_specs=in_specs,
        out_specs=pl.BlockSpec((n, c_out), lambda i: (0, 0)),
        out_shape=jax.ShapeDtypeStruct((n, c_out), jnp.float32),
    )(*ins)
    return out


# speedup vs baseline: 3.4234x; 1.1455x over previous
import functools

import jax
import jax.numpy as jnp
from jax import lax
from jax.experimental import pallas as pl
from jax.experimental.pallas import tpu as pltpu
from jax.experimental.pallas import tpu_sc as plsc

NC = 2
NS = 16
NW = NC * NS
CHUNK = 128
INNER = 2
NG = 1
M0 = 54
M1 = 26


def _matmul_body(f_ref, w_ref, o_ref):
    o_ref[0] = jnp.dot(f_ref[...], w_ref[0], preferred_element_type=jnp.float32)


def _sum_relu_body(*refs):
    o_ref = refs[-1]
    acc = refs[0][0]
    for r in refs[1:-1]:
        acc = acc + r[0]
    o_ref[...] = jnp.maximum(acc, 0.0)


@functools.partial(jax.jit, static_argnames=("npad", "c", "rpt"))
def _sc_gather_scatter(idx, tflat, zeros, *, npad, c, rpt):
    mesh = plsc.VectorSubcoreMesh(
        core_axis_name="c", subcore_axis_name="s", num_cores=NC, num_subcores=NS
    )

    def body(idx_hbm, tfeats_hbm, zeros_hbm, out_hbm,
             idx_v, rows_v, accum_sh, isem, gsem, ssem):
        cid = lax.axis_index("c")
        sid = lax.axis_index("s")
        wid = cid * NS + sid
        pltpu.sync_copy(idx_hbm.at[wid, 0], idx_v.at[0])
        pltpu.sync_copy(zeros_hbm.at[pl.ds(sid * rpt, rpt)],
                        accum_sh.at[pl.ds(sid * rpt, rpt)])
        plsc.subcore_barrier()

        def outer_pair(t2, carry):
            for p in range(2):
                t = 2 * t2 + p
                nxt = pltpu.async_copy(idx_hbm.at[wid, t + 1],
                                       idx_v.at[1 - p], isem)
                gd = [
                    pltpu.async_copy(tfeats_hbm.at[idx_v.at[p, 0, b]],
                                     rows_v.at[b], gsem)
                    for b in range(INNER)
                ]
                sd = []
                for b in range(INNER):
                    gd[b].wait()
                    sd.append(pltpu.async_copy(rows_v.at[b],
                                               accum_sh.at[idx_v.at[p, 1, b]],
                                               ssem, add=True))
                for d in sd:
                    d.wait()
                nxt.wait()
            return carry

        lax.fori_loop(0, lax.select(cid == 0, M0 // 2, M1 // 2),
                      outer_pair, 0)
        plsc.subcore_barrier()
        pltpu.sync_copy(accum_sh.at[pl.ds(sid * rpt, rpt)],
                        out_hbm.at[cid, pl.ds(sid * rpt, rpt)])

    return pl.kernel(
        body,
        out_type=jax.ShapeDtypeStruct((NC, npad, c), jnp.float32),
        mesh=mesh,
        scratch_types=[
            pltpu.VMEM((2, 2, INNER, CHUNK), jnp.int32),
            pltpu.VMEM((INNER, CHUNK, c), jnp.float32),
            pltpu.VMEM_SHARED((npad, c), jnp.float32),
            pltpu.SemaphoreType.DMA,
            pltpu.SemaphoreType.DMA,
            pltpu.SemaphoreType.DMA,
        ],
    )(idx, tflat, zeros)


def kernel(feats, in_map, out_map, W):
    n, c_in = feats.shape
    k, e = in_map.shape
    c_out = W.shape[-1]
    kg = k // NG
    assert kg * NG == k

    in32 = in_map.astype(jnp.int32)
    out32 = out_map.astype(jnp.int32)
    feats_bf = feats.astype(jnp.bfloat16)
    w_bf = W.astype(jnp.bfloat16)

    blk = INNER * CHUNK
    ep0 = NS * M0 * blk
    ep1 = NS * M1 * blk
    ep = ep0 + ep1
    eg = kg * e
    assert ep >= eg
    npad = ((n + NS + 127) // 128) * 128
    rpt = npad // NS
    zeros = jnp.zeros((npad, c_out), jnp.float32)
    bn = 2000

    def _pack(x):
        p0 = x[:ep0].reshape(NS, M0, 1, INNER, CHUNK)
        p1 = x[ep0:].reshape(NS, M1, 1, INNER, CHUNK)
        pad0 = jnp.zeros((NS, 1, 1, INNER, CHUNK), jnp.int32)
        pad1 = jnp.zeros((NS, M0 - M1 + 1, 1, INNER, CHUNK), jnp.int32)
        return jnp.concatenate(
            [jnp.concatenate([p0, pad0], axis=1),
             jnp.concatenate([p1, pad1], axis=1)], axis=0)

    offs = (jnp.arange(kg, dtype=jnp.int32) * n)[:, None]
    partials = []
    for g in range(NG):
        tfeats = pl.pallas_call(
            _matmul_body,
            grid=(n // bn, kg),
            in_specs=[
                pl.BlockSpec((bn, c_in), lambda ni, ki: (ni, 0)),
                pl.BlockSpec((1, c_in, c_out), lambda ni, ki: (ki, 0, 0)),
            ],
            out_specs=pl.BlockSpec((1, bn, c_out), lambda ni, ki: (ki, ni, 0)),
            out_shape=jax.ShapeDtypeStruct((kg, n, c_out), jnp.float32),
        )(feats_bf, w_bf[g * kg:(g + 1) * kg])
        tflat = tfeats.reshape(kg * n, c_out)

        fi = (in32[g * kg:(g + 1) * kg] + offs).reshape(-1)
        fo = out32[g * kg:(g + 1) * kg].reshape(-1)
        fi = jnp.concatenate([fi, jnp.zeros((ep - eg,), jnp.int32)])
        fo = jnp.concatenate([fo, jnp.full((ep - eg,), n, jnp.int32)])
        idx = jnp.concatenate([_pack(fi), _pack(fo)], axis=2)

        partials.append(_sc_gather_scatter(
            idx, tflat, zeros, npad=npad, c=c_out, rpt=rpt))

    ins = []
    in_specs = []
    for p in partials:
        for cc in range(NC):
            ins.append(p)
            in_specs.append(
                pl.BlockSpec((1, n, c_out), lambda i, cc=cc: (cc, 0, 0)))
    out = pl.pallas_call(
        _sum_relu_body,
        grid=(1,),
        in_specs=in_specs,
        out_specs=pl.BlockSpec((n, c_out), lambda i: (0, 0)),
        out_shape=jax.ShapeDtypeStruct((n, c_out), jnp.float32),
    )(*ins)
    return out
